# baseline (speedup 1.0000x reference)
import math

import jax
import jax.numpy as jnp
from jax.experimental import pallas as pl
from jax.experimental.pallas import tpu as pltpu

H = 16
NBUC = 32
QL = 2048
KL = 2048
QH = 40
RB = 128


def _flat_shift(x, n):
    rl = pltpu.roll(x, 128 - n, axis=2)
    sub = pltpu.roll(rl, QH - 1, axis=1)
    lane = jax.lax.broadcasted_iota(jnp.int32, x.shape, 2)
    return jnp.where(lane < 128 - n, rl, sub)


def _body(delta_ref, table_t_ref, out_ref, vals_ref, bank_ref):
    h = pl.program_id(0)
    g = pl.program_id(1)

    @pl.when((h == 0) & (g == 0))
    def _compute_vals():
        q = jax.lax.broadcasted_iota(jnp.int32, (H, QH, 128), 1)
        l = jax.lax.broadcasted_iota(jnp.int32, (H, QH, 128), 2)
        d = 128 * q + l - (QL - 1) + delta_ref[0]
        rb = jnp.where(d > 0, 16, 0).astype(jnp.int32)
        a = jnp.abs(d)
        is_small = a < 8
        rp_safe = jnp.maximum(a, 1)
        large = 8 + (
            jnp.log(rp_safe.astype(jnp.float32) / 8)
            / math.log(128 / 8)
            * (16 - 8)
        ).astype(jnp.int32)
        large = jnp.minimum(large, jnp.full_like(large, 15))
        bucket = rb + jnp.where(is_small, a, large)
        acc = jnp.zeros((H, QH, 128), jnp.float32)
        for b in range(NBUC):
            acc = jnp.where(bucket == b, table_t_ref[:, pl.ds(b, 1)][:, :, None], acc)
        vals_ref[...] = acc

    @pl.when(g == 0)
    def _build_bank():
        bank_ref[RB - 1, :, :] = vals_ref[h]
        for k in range(7):
            n = 1 << k
            src = bank_ref[RB - n : RB, :, :]
            bank_ref[RB - 2 * n : RB - n, :, :] = _flat_shift(src, n)

    out_ref[0] = bank_ref[:, pl.ds(15 - g, 16), :]


def kernel(query_length, key_length, relative_attention_bias):
    delta = (
        (jnp.asarray(key_length, jnp.int32) - KL)
        - (jnp.asarray(query_length, jnp.int32) - QL)
    ).reshape(1)
    table_t = relative_attention_bias.T
    out = pl.pallas_call(
        _body,
        grid=(H, QL // RB),
        in_specs=[
            pl.BlockSpec(memory_space=pltpu.SMEM),
            pl.BlockSpec((H, NBUC), lambda h, g: (0, 0)),
        ],
        out_specs=pl.BlockSpec((1, RB, 16, 128), lambda h, g: (h, g, 0, 0)),
        out_shape=jax.ShapeDtypeStruct((H, QL, 16, 128), jnp.float32),
        scratch_shapes=[
            pltpu.VMEM((H, QH, 128), jnp.float32),
            pltpu.VMEM((RB, QH, 128), jnp.float32),
        ],
    )(delta, table_t)
    return out.reshape(1, H, QL, KL)


# speedup vs baseline: 2.6326x; 2.6326x over previous
import math

import jax
import jax.numpy as jnp
from jax.experimental import pallas as pl
from jax.experimental.pallas import tpu as pltpu

H = 16
NBUC = 32
QL = 2048
KL = 2048
QH = 40
RB = 128


def _body(delta_ref, table_t_ref, out_ref, vals_ref, bank_ref):
    h = pl.program_id(0)
    g = pl.program_id(1)

    @pl.when((h == 0) & (g == 0))
    def _compute_vals():
        q = jax.lax.broadcasted_iota(jnp.int32, (H, QH, 128), 1)
        l = jax.lax.broadcasted_iota(jnp.int32, (H, QH, 128), 2)
        d = 128 * q + l - (QL - 1) + delta_ref[0]
        rb = jnp.where(d > 0, 16, 0).astype(jnp.int32)
        a = jnp.abs(d)
        is_small = a < 8
        rp_safe = jnp.maximum(a, 1)
        large = 8 + (
            jnp.log(rp_safe.astype(jnp.float32) / 8)
            / math.log(128 / 8)
            * (16 - 8)
        ).astype(jnp.int32)
        large = jnp.minimum(large, jnp.full_like(large, 15))
        bucket = rb + jnp.where(is_small, a, large)
        acc = jnp.zeros((H, QH, 128), jnp.float32)
        for b in range(NBUC):
            acc = jnp.where(bucket == b, table_t_ref[:, pl.ds(b, 1)][:, :, None], acc)
        vals_ref[...] = acc

    @pl.when(g == 0)
    def _build_bank():
        bank_ref[:, RB - 1, :] = vals_ref[h]
        for k in range(7):
            n = 1 << k
            src = bank_ref[:, RB - n : RB, :]
            rl = pltpu.roll(src, 128 - n, axis=2)
            sub = jnp.concatenate([rl[1:], rl[:1]], axis=0)
            lane = jax.lax.broadcasted_iota(jnp.int32, (QH, n, 128), 2)
            bank_ref[:, RB - 2 * n : RB - n, :] = jnp.where(lane < 128 - n, rl, sub)

    for c in range(16):
        out_ref[0, 0, :, 128 * c : 128 * (c + 1)] = bank_ref[15 - g + c]


def kernel(query_length, key_length, relative_attention_bias):
    delta = (
        (jnp.asarray(key_length, jnp.int32) - KL)
        - (jnp.asarray(query_length, jnp.int32) - QL)
    ).reshape(1)
    table_t = relative_attention_bias.T
    return pl.pallas_call(
        _body,
        grid=(H, QL // RB),
        in_specs=[
            pl.BlockSpec(memory_space=pltpu.SMEM),
            pl.BlockSpec((H, NBUC), lambda h, g: (0, 0)),
        ],
        out_specs=pl.BlockSpec((1, 1, RB, KL), lambda h, g: (0, h, g, 0)),
        out_shape=jax.ShapeDtypeStruct((1, H, QL, KL), jnp.float32),
        scratch_shapes=[
            pltpu.VMEM((H, QH, 128), jnp.float32),
            pltpu.VMEM((QH, RB, 128), jnp.float32),
        ],
    )(delta, table_t)


# speedup vs baseline: 2.9610x; 1.1247x over previous
import functools
import math

import jax
import jax.numpy as jnp
from jax import lax
from jax.experimental import pallas as pl
from jax.experimental.pallas import tpu as pltpu
from jax.experimental.pallas import tpu_sc as plsc

H = 16
NBUC = 32
QL = 2048
KL = 2048
QH = 40
NP = 128
SW = 31 * 128
NIH = QL // 8


def _pat_body(delta_ref, table_t_ref, pat_ref, vals_ref, bank_ref):
    h = pl.program_id(0)

    @pl.when(h == 0)
    def _compute_vals():
        q = jax.lax.broadcasted_iota(jnp.int32, (H, QH, 128), 1)
        l = jax.lax.broadcasted_iota(jnp.int32, (H, QH, 128), 2)
        d = 128 * q + l - (QL - 1) + delta_ref[0]
        rb = jnp.where(d > 0, 16, 0).astype(jnp.int32)
        a = jnp.abs(d)
        is_small = a < 8
        rp_safe = jnp.maximum(a, 1)
        large = 8 + (
            jnp.log(rp_safe.astype(jnp.float32) / 8)
            / math.log(128 / 8)
            * (16 - 8)
        ).astype(jnp.int32)
        large = jnp.minimum(large, jnp.full_like(large, 15))
        bucket = rb + jnp.where(is_small, a, large)
        acc = jnp.zeros((H, QH, 128), jnp.float32)
        for b in range(NBUC):
            acc = jnp.where(bucket == b, table_t_ref[:, pl.ds(b, 1)][:, :, None], acc)
        vals_ref[...] = acc

    bank_ref[:, NP - 1, :] = vals_ref[h]
    for k in range(7):
        n = 1 << k
        src = bank_ref[:, NP - n : NP, :]
        rl = pltpu.roll(src, 128 - n, axis=2)
        sub = jnp.concatenate([rl[1:], rl[:1]], axis=0)
        lane = jax.lax.broadcasted_iota(jnp.int32, (QH, n, 128), 2)
        bank_ref[:, NP - 2 * n : NP - n, :] = jnp.where(lane < 128 - n, rl, sub)

    for a in range(16):
        for q in range(SW // 128):
            pat_ref[0, a, :, 128 * q : 128 * (q + 1)] = bank_ref[
                q, 8 * (15 - a) : 8 * (15 - a) + 8, :
            ]


def _sc_body(pat_hbm, out_hbm, slab, sem):
    wid = lax.axis_index("s") * 2 + lax.axis_index("c")
    for e in range(8):
        idx = wid * 8 + e
        head = idx // 16
        r = idx % 16
        pltpu.sync_copy(pat_hbm.at[head, 15 - r], slab)
        copies = []
        for t in range(16):
            i_hi = r + 16 * t
            copies.append(
                pltpu.async_copy(
                    slab.at[:, pl.ds(128 * (15 - t), KL)],
                    out_hbm.at[head, i_hi],
                    sem,
                )
            )
        for cp in copies:
            cp.wait()


def kernel(query_length, key_length, relative_attention_bias):
    delta = (
        (jnp.asarray(key_length, jnp.int32) - KL)
        - (jnp.asarray(query_length, jnp.int32) - QL)
    ).reshape(1)
    table_t = relative_attention_bias.T
    pat = pl.pallas_call(
        _pat_body,
        grid=(H,),
        in_specs=[
            pl.BlockSpec(memory_space=pltpu.SMEM),
            pl.BlockSpec((H, NBUC), lambda h: (0, 0)),
        ],
        out_specs=pl.BlockSpec((1, 16, 8, SW), lambda h: (h, 0, 0, 0)),
        out_shape=jax.ShapeDtypeStruct((H, 16, 8, SW), jnp.float32),
        scratch_shapes=[
            pltpu.VMEM((H, QH, 128), jnp.float32),
            pltpu.VMEM((QH, NP, 128), jnp.float32),
        ],
    )(delta, table_t)

    sc_call = functools.partial(
        pl.kernel,
        out_type=jax.ShapeDtypeStruct((H, NIH, 8, KL), jnp.float32),
        mesh=plsc.VectorSubcoreMesh(core_axis_name="c", subcore_axis_name="s"),
        scratch_types=[
            pltpu.VMEM((8, SW), jnp.float32),
            pltpu.SemaphoreType.DMA,
        ],
    )(_sc_body)
    out = sc_call(pat)
    return out.reshape(1, H, QL, KL)


# speedup vs baseline: 3.0095x; 1.0164x over previous
import functools
import math

import jax
import jax.numpy as jnp
from jax import lax
from jax.experimental import pallas as pl
from jax.experimental.pallas import tpu as pltpu
from jax.experimental.pallas import tpu_sc as plsc

H = 16
NBUC = 32
QL = 2048
KL = 2048
QH = 40
NP = 128
SW = 31 * 128
NIH = QL // 8


def _pat_body(delta_ref, table_t_ref, pat_ref, vals_ref, bank_ref):
    h = pl.program_id(0)

    @pl.when(h == 0)
    def _compute_vals():
        q = jax.lax.broadcasted_iota(jnp.int32, (H, QH, 128), 1)
        l = jax.lax.broadcasted_iota(jnp.int32, (H, QH, 128), 2)
        d = 128 * q + l - (QL - 1) + delta_ref[0]
        rb = jnp.where(d > 0, 16, 0).astype(jnp.int32)
        a = jnp.abs(d)
        is_small = a < 8
        rp_safe = jnp.maximum(a, 1)
        large = 8 + (
            jnp.log(rp_safe.astype(jnp.float32) / 8)
            / math.log(128 / 8)
            * (16 - 8)
        ).astype(jnp.int32)
        large = jnp.minimum(large, jnp.full_like(large, 15))
        bucket = rb + jnp.where(is_small, a, large)
        acc = jnp.zeros((H, QH, 128), jnp.float32)
        for b in range(NBUC):
            acc = jnp.where(bucket == b, table_t_ref[:, pl.ds(b, 1)][:, :, None], acc)
        vals_ref[...] = acc

    bank_ref[:, NP - 1, :] = vals_ref[h]
    for k in range(7):
        n = 1 << k
        src = bank_ref[:, NP - n : NP, :]
        rl = pltpu.roll(src, 128 - n, axis=2)
        sub = jnp.concatenate([rl[1:], rl[:1]], axis=0)
        lane = jax.lax.broadcasted_iota(jnp.int32, (QH, n, 128), 2)
        bank_ref[:, NP - 2 * n : NP - n, :] = jnp.where(lane < 128 - n, rl, sub)

    for a in range(16):
        for q in range(SW // 128):
            pat_ref[0, a, :, 128 * q : 128 * (q + 1)] = bank_ref[
                q, 8 * (15 - a) : 8 * (15 - a) + 8, :
            ]


def _sc_body(pat_hbm, out_hbm, slab, sem, psem):
    wid = lax.axis_index("s") * 2 + lax.axis_index("c")

    def stage(e, buf):
        idx = wid * 8 + e
        return pltpu.async_copy(
            pat_hbm.at[idx // 16, 15 - (idx % 16)], slab.at[buf], psem
        )

    stage(0, 0).wait()
    for e in range(8):
        cur = e % 2
        if e < 7:
            prefetch = stage(e + 1, 1 - cur)
        idx = wid * 8 + e
        head = idx // 16
        r = idx % 16
        copies = []
        for t in range(16):
            i_hi = r + 16 * t
            copies.append(
                pltpu.async_copy(
                    slab.at[cur, :, pl.ds(128 * (15 - t), KL)],
                    out_hbm.at[head, i_hi],
                    sem,
                )
            )
        for cp in copies:
            cp.wait()
        if e < 7:
            prefetch.wait()


def kernel(query_length, key_length, relative_attention_bias):
    delta = (
        (jnp.asarray(key_length, jnp.int32) - KL)
        - (jnp.asarray(query_length, jnp.int32) - QL)
    ).reshape(1)
    table_t = relative_attention_bias.T
    pat = pl.pallas_call(
        _pat_body,
        grid=(H,),
        in_specs=[
            pl.BlockSpec(memory_space=pltpu.SMEM),
            pl.BlockSpec((H, NBUC), lambda h: (0, 0)),
        ],
        out_specs=pl.BlockSpec((1, 16, 8, SW), lambda h: (h, 0, 0, 0)),
        out_shape=jax.ShapeDtypeStruct((H, 16, 8, SW), jnp.float32),
        scratch_shapes=[
            pltpu.VMEM((H, QH, 128), jnp.float32),
            pltpu.VMEM((QH, NP, 128), jnp.float32),
        ],
    )(delta, table_t)

    sc_call = functools.partial(
        pl.kernel,
        out_type=jax.ShapeDtypeStruct((H, NIH, 8, KL), jnp.float32),
        mesh=plsc.VectorSubcoreMesh(core_axis_name="c", subcore_axis_name="s"),
        scratch_types=[
            pltpu.VMEM((2, 8, SW), jnp.float32),
            pltpu.SemaphoreType.DMA,
            pltpu.SemaphoreType.DMA,
        ],
    )(_sc_body)
    out = sc_call(pat)
    return out.reshape(1, H, QL, KL)
